# baseline (speedup 1.0000x reference)
import functools

import jax
import jax.numpy as jnp
from jax import lax
from jax.experimental import pallas as pl
from jax.experimental.pallas import tpu as pltpu
from jax.experimental.pallas import tpu_sc as plsc

EPS = 1e-6
BLK = 256
BLKF = 128
NC, NS, L = 2, 16, 16


def _rot_perm(hd):
    h = hd // 2
    eye = jnp.eye(h, dtype=jnp.float32)
    z = jnp.zeros((h, h), jnp.float32)
    return jnp.block([[z, eye], [-eye, z]])


def _prelude_body(x_ref, anw_ref, wq_ref, wk_ref, wv_ref, qnw_ref, knw_ref,
                  cq_ref, sq_ref, ck_ref, sk_ref, pq_ref, pk_ref,
                  hq_ref, hqt_ref, hk_ref, hkt_ref,
                  q_ref, k_ref, v_ref, *, hd):
    xs = x_ref[...]
    a = xs * jax.lax.rsqrt(jnp.mean(xs * xs, axis=-1, keepdims=True) + EPS)
    a = (a * anw_ref[...]).astype(jnp.bfloat16)

    def qk_path(w_ref, nw_ref, h_ref, ht_ref, p_ref, c_ref, s_ref):
        q = jnp.dot(a, w_ref[...], preferred_element_type=jnp.float32)
        ss = jnp.dot(q * q, h_ref[...], preferred_element_type=jnp.float32)
        rs = jax.lax.rsqrt(ss / hd + EPS)
        qn = q * jnp.dot(rs, ht_ref[...], preferred_element_type=jnp.float32)
        qn = qn * nw_ref[...]
        qr = jnp.dot(qn.astype(jnp.bfloat16), p_ref[...],
                     preferred_element_type=jnp.float32)
        return (qn * c_ref[...] + qr * s_ref[...]).astype(jnp.bfloat16)

    q_ref[...] = qk_path(wq_ref, qnw_ref, hq_ref, hqt_ref, pq_ref, cq_ref, sq_ref)
    k_ref[...] = qk_path(wk_ref, knw_ref, hk_ref, hkt_ref, pk_ref, ck_ref, sk_ref)
    v_ref[...] = jnp.dot(a, wv_ref[...],
                         preferred_element_type=jnp.float32).astype(jnp.bfloat16)


def _attn_body(q_ref, k_ref, v_ref, o_ref, *, hd):
    s = jax.lax.dot_general(q_ref[0], k_ref[0],
                            (((1,), (1,)), ((), ())),
                            preferred_element_type=jnp.float32)
    s = s * (1.0 / (hd ** 0.5))
    m = jnp.max(s, axis=-1, keepdims=True)
    e = jnp.exp(s - m)
    p = e / jnp.sum(e, axis=-1, keepdims=True)
    o_ref[0] = jnp.dot(p.astype(jnp.bfloat16), v_ref[0],
                       preferred_element_type=jnp.float32).astype(jnp.bfloat16)


def _post_body(ctx_ref, wo_ref, x_ref, fnw_ref, rw_ref,
               x2_ref, m_ref, oa_ref, ob_ref, wa_ref, wb_ref, *, ne):
    x2 = x_ref[...] + jnp.dot(ctx_ref[...], wo_ref[...],
                              preferred_element_type=jnp.float32)
    x2_ref[...] = x2
    mm = x2 * jax.lax.rsqrt(jnp.mean(x2 * x2, axis=-1, keepdims=True) + EPS)
    mm = mm * fnw_ref[...]
    m_ref[...] = mm.astype(jnp.bfloat16)
    logits = jnp.dot(mm, rw_ref[...], preferred_element_type=jnp.float32)
    mx = jnp.max(logits, axis=-1, keepdims=True)
    ex = jnp.exp(logits - mx)
    g = ex / jnp.sum(ex, axis=-1, keepdims=True)
    it = jax.lax.broadcasted_iota(jnp.int32, g.shape, 1)
    m1 = jnp.max(g, axis=-1, keepdims=True)
    i1 = jnp.min(jnp.where(g == m1, it, ne), axis=-1, keepdims=True)
    g2 = jnp.where(it == i1, -jnp.inf, g)
    m2 = jnp.max(g2, axis=-1, keepdims=True)
    i2 = jnp.min(jnp.where(g2 == m2, it, ne), axis=-1, keepdims=True)
    oa = (it == i1)
    ob = (it == i2)
    oa_ref[...] = oa.astype(jnp.float32)
    ob_ref[...] = ob.astype(jnp.float32)
    wa_ref[...] = m1[:, 0]
    wb_ref[...] = m2[:, 0]


def _route_body(oa_ref, ob_ref, ls_ref, posa_ref, posb_ref, be_ref,
                *, ne, nblk):
    oa = oa_ref[...]
    ob = ob_ref[...]
    t = oa.shape[0]
    oab = oa.astype(jnp.bfloat16)
    obb = ob.astype(jnp.bfloat16)
    ls = ls_ref[...]
    ra = jnp.dot(ls, oab, preferred_element_type=jnp.float32)
    rb = jnp.dot(ls, obb, preferred_element_type=jnp.float32)
    tot_a = jnp.sum(oa, axis=0, keepdims=True)
    cnt = tot_a + jnp.sum(ob, axis=0, keepdims=True)
    blocks = jnp.floor((cnt + (BLKF - 1)) * (1.0 / BLKF))
    eiota_r = jax.lax.broadcasted_iota(jnp.int32, (ne, ne), 0)
    eiota_c = jax.lax.broadcasted_iota(jnp.int32, (ne, ne), 1)
    m8 = (eiota_r < eiota_c).astype(jnp.float32)
    sblk = jnp.dot(blocks, m8, preferred_element_type=jnp.float32)
    spad = sblk * BLKF
    posa = jnp.sum(oa * (ra + spad), axis=1)
    posb = jnp.sum(ob * (rb + tot_a + spad), axis=1)
    posa_ref[...] = posa.astype(jnp.int32)
    posb_ref[...] = posb.astype(jnp.int32)
    cnt_t = jax.lax.dot_general(oa + ob, jnp.ones((t, 1), jnp.float32),
                                (((0,), (0,)), ((), ())),
                                preferred_element_type=jnp.float32)
    blocks_t = jnp.floor((cnt_t + (BLKF - 1)) * (1.0 / BLKF))
    m8l = (eiota_c < eiota_r).astype(jnp.float32)
    sblk_t = jnp.dot(m8l, blocks_t, preferred_element_type=jnp.float32)
    biota = jax.lax.broadcasted_iota(jnp.int32, (ne, nblk), 1).astype(jnp.float32)
    ge = (biota >= sblk_t).astype(jnp.float32)
    be = jnp.sum(ge, axis=0, keepdims=True) - 1.0
    total = jnp.sum(blocks_t)
    biota1 = jax.lax.broadcasted_iota(jnp.int32, (1, nblk), 1).astype(jnp.float32)
    be = jnp.where(biota1 < total, be, -1.0)
    be_ref[...] = be[0].astype(jnp.int32)


def _gffn_body(be_ref, xg_ref, wg_ref, wi_ref, woe_ref, rg_ref, y_ref):
    b = pl.program_id(0)

    @pl.when(be_ref[b] >= 0)
    def _compute():
        mb = xg_ref[...]
        g = jnp.dot(mb, wg_ref[0], preferred_element_type=jnp.float32)
        u = jnp.dot(mb, wi_ref[0], preferred_element_type=jnp.float32)
        h = (g * jax.nn.sigmoid(g) * u).astype(jnp.bfloat16)
        y = jnp.dot(h, woe_ref[0], preferred_element_type=jnp.float32)
        w = jnp.reshape(rg_ref[...], (y.shape[0], 1))
        y_ref[...] = (y * w).astype(jnp.bfloat16)


def _sc_mesh():
    return plsc.VectorSubcoreMesh(core_axis_name="c", subcore_axis_name="s",
                                  num_cores=NC, num_subcores=NS)


def _sc_dispatch(pos_all, gate_all, m, *, nrows, rpt, t, dim, nslots):
    f32 = jnp.float32
    dim2 = dim // 2
    m32 = jax.lax.bitcast_convert_type(m.reshape(t, dim2, 2), jnp.uint32)
    gchunk = min(80, rpt)

    @functools.partial(
        pl.kernel,
        out_type=[jax.ShapeDtypeStruct((nrows, dim2), jnp.uint32),
                  jax.ShapeDtypeStruct((nrows,), f32)],
        mesh=_sc_mesh(),
        scratch_types=[pltpu.VMEM((nslots,), jnp.int32),
                       pltpu.VMEM((nslots,), f32),
                       pltpu.VMEM((rpt,), jnp.int32),
                       pltpu.VMEM((rpt,), f32),
                       pltpu.VMEM((rpt, dim2), jnp.uint32),
                       pltpu.SemaphoreType.DMA],
        compiler_params=pltpu.CompilerParams(needs_layout_passes=False),
    )
    def _dispatch(pos_hbm, gate_hbm, m_hbm, xg_hbm, rg_hbm,
                  pos_v, gate_v, gidx_v, rg_v, rows_v, sem):
        wid = lax.axis_index("s") * NC + lax.axis_index("c")
        base = wid * rpt
        pltpu.sync_copy(pos_hbm, pos_v)
        pltpu.sync_copy(gate_hbm, gate_v)
        zi = jnp.zeros((L,), jnp.int32)
        zf = jnp.zeros((L,), f32)
        for i in range(rpt // L):
            gidx_v[pl.ds(i * L, L)] = zi
            rg_v[pl.ds(i * L, L)] = zf

        def body(g, carry):
            pv = pos_v[pl.ds(g * L, L)]
            rel = pv - base
            msk = (rel >= 0) & (rel < rpt)
            relc = jnp.where(msk, rel, 0)
            tok = (g * L + lax.iota(jnp.int32, L)) & (t - 1)
            plsc.store_scatter(gidx_v, [relc], tok, mask=msk)
            gv = gate_v[pl.ds(g * L, L)]
            plsc.store_scatter(rg_v, [relc], gv, mask=msk)
            return carry

        lax.fori_loop(0, nslots // L, body, 0)
        for c in range(0, rpt, gchunk):
            pltpu.async_copy(m_hbm.at[gidx_v.at[pl.ds(c, gchunk)]],
                             rows_v.at[pl.ds(c, gchunk)], sem).wait()
        pltpu.sync_copy(rows_v, xg_hbm.at[pl.ds(base, rpt)])
        pltpu.sync_copy(rg_v, rg_hbm.at[pl.ds(base, rpt)])

    xg32, rgate = _dispatch(pos_all, gate_all, m32)
    xg = jax.lax.bitcast_convert_type(xg32, jnp.bfloat16).reshape(nrows, dim)
    return xg, rgate


def _sc_combine(pos_all, y, *, spt, dim, nslots):
    nry = y.shape[0]
    dim2 = dim // 2
    y32 = jax.lax.bitcast_convert_type(y.reshape(nry, dim2, 2), jnp.uint32)

    @functools.partial(
        pl.kernel,
        out_type=jax.ShapeDtypeStruct((nslots, dim2), jnp.uint32),
        mesh=_sc_mesh(),
        scratch_types=[pltpu.VMEM((spt,), jnp.int32),
                       pltpu.VMEM((spt, dim2), jnp.uint32),
                       pltpu.SemaphoreType.DMA],
        compiler_params=pltpu.CompilerParams(needs_layout_passes=False),
    )
    def _combine(pos_hbm, y_hbm, yg_hbm, idx_v, rows_v, sem):
        wid = lax.axis_index("s") * NC + lax.axis_index("c")
        base = wid * spt
        pltpu.sync_copy(pos_hbm.at[pl.ds(base, spt)], idx_v)
        pltpu.async_copy(y_hbm.at[idx_v], rows_v, sem).wait()
        pltpu.sync_copy(rows_v, yg_hbm.at[pl.ds(base, spt)])

    yg32 = _combine(pos_all, y32)
    return jax.lax.bitcast_convert_type(yg32, jnp.bfloat16).reshape(nslots, dim)


def _final_body(x2_ref, ya_ref, yb_ref, o_ref):
    o_ref[...] = (x2_ref[...] + ya_ref[...].astype(jnp.float32)
                  + yb_ref[...].astype(jnp.float32))


def kernel(x, attn_norm_w, Wq, Wk, Wv, Wo, q_norm_w, k_norm_w, ffn_norm_w,
           Wi, Wg, Woe, router_w, cos, sin):
    b, t, dim = x.shape
    nq = Wq.shape[1] // cos.shape[1]
    nkv = Wk.shape[1] // cos.shape[1]
    hd = cos.shape[1]
    ne, _, hid = Wi.shape
    blk = min(BLK, t)
    nt = t // blk
    nslots = 2 * t
    nblk = nslots // BLKF + ne
    nrows = nblk * BLKF
    nw = NC * NS
    spt = nslots // nw
    rpt = nrows // nw

    x2d = x.reshape(t, dim)
    bf = jnp.bfloat16
    f32 = jnp.float32
    wq_b, wk_b, wv_b, wo_b = (w.astype(bf) for w in (Wq, Wk, Wv, Wo))
    wi_b, wg_b, woe_b = (w.astype(bf) for w in (Wi, Wg, Woe))

    p64 = _rot_perm(hd)
    pq = jnp.kron(jnp.eye(nq, dtype=f32), p64).astype(bf)
    pk = jnp.kron(jnp.eye(nkv, dtype=f32), p64).astype(bf)
    hq = jnp.kron(jnp.eye(nq, dtype=f32), jnp.ones((hd, 1), f32))
    hk = jnp.kron(jnp.eye(nkv, dtype=f32), jnp.ones((hd, 1), f32))
    cq = jnp.tile(cos, (1, nq))
    sq = jnp.tile(sin, (1, nq))
    ck = jnp.tile(cos, (1, nkv))
    sk = jnp.tile(sin, (1, nkv))
    qnw = jnp.tile(q_norm_w, (nq,)).reshape(1, nq * hd)
    knw = jnp.tile(k_norm_w, (nkv,)).reshape(1, nkv * hd)
    anw = attn_norm_w.reshape(1, dim)
    fnw = ffn_norm_w.reshape(1, dim)
    tio_r = jax.lax.broadcasted_iota(jnp.int32, (t, t), 0)
    tio_c = jax.lax.broadcasted_iota(jnp.int32, (t, t), 1)
    ls2048 = (tio_c < tio_r).astype(bf)

    dq, dkv = nq * hd, nkv * hd

    full = lambda shape: pl.BlockSpec(shape, lambda i: (0,) * len(shape))
    rowblk = lambda w: pl.BlockSpec((blk, w), lambda i: (i, 0))

    q, k, v = pl.pallas_call(
        functools.partial(_prelude_body, hd=hd),
        grid=(nt,),
        in_specs=[
            rowblk(dim), full((1, dim)), full((dim, dq)), full((dim, dkv)),
            full((dim, dkv)), full((1, dq)), full((1, dkv)),
            rowblk(dq), rowblk(dq), rowblk(dkv), rowblk(dkv),
            full((dq, dq)), full((dkv, dkv)),
            full((dq, nq)), full((nq, dq)), full((dkv, nkv)), full((nkv, dkv)),
        ],
        out_specs=[rowblk(dq), rowblk(dkv), rowblk(dkv)],
        out_shape=[
            jax.ShapeDtypeStruct((t, dq), bf),
            jax.ShapeDtypeStruct((t, dkv), bf),
            jax.ShapeDtypeStruct((t, dkv), bf),
        ],
    )(x2d, anw, wq_b, wk_b, wv_b, qnw, knw, cq, sq, ck, sk,
      pq, pk, hq, hq.T, hk, hk.T)

    rep = nq // nkv
    q3 = q.reshape(t, nq, hd).transpose(1, 0, 2)
    k3 = k.reshape(t, nkv, hd).transpose(1, 0, 2)
    v3 = v.reshape(t, nkv, hd).transpose(1, 0, 2)
    ctx3 = pl.pallas_call(
        functools.partial(_attn_body, hd=hd),
        grid=(nq, nt),
        in_specs=[
            pl.BlockSpec((1, blk, hd), lambda h, i: (h, i, 0)),
            pl.BlockSpec((1, t, hd), lambda h, i: (h // rep, 0, 0)),
            pl.BlockSpec((1, t, hd), lambda h, i: (h // rep, 0, 0)),
        ],
        out_specs=pl.BlockSpec((1, blk, hd), lambda h, i: (h, i, 0)),
        out_shape=jax.ShapeDtypeStruct((nq, t, hd), bf),
    )(q3, k3, v3)
    ctx = ctx3.transpose(1, 0, 2).reshape(t, dq)

    x2, m, oa, ob, wa, wb = pl.pallas_call(
        functools.partial(_post_body, ne=ne),
        grid=(nt,),
        in_specs=[rowblk(dq), full((dq, dim)), rowblk(dim), full((1, dim)),
                  full((dim, ne))],
        out_specs=[rowblk(dim), rowblk(dim), rowblk(ne), rowblk(ne),
                   pl.BlockSpec((blk,), lambda i: (i,)),
                   pl.BlockSpec((blk,), lambda i: (i,))],
        out_shape=[
            jax.ShapeDtypeStruct((t, dim), f32),
            jax.ShapeDtypeStruct((t, dim), bf),
            jax.ShapeDtypeStruct((t, ne), f32),
            jax.ShapeDtypeStruct((t, ne), f32),
            jax.ShapeDtypeStruct((t,), f32),
            jax.ShapeDtypeStruct((t,), f32),
        ],
    )(ctx, wo_b, x2d, fnw, router_w)

    posa, posb, be = pl.pallas_call(
        functools.partial(_route_body, ne=ne, nblk=nblk),
        grid=(1,),
        in_specs=[full((t, ne)), full((t, ne)), full((t, t))],
        out_specs=[pl.BlockSpec((t,), lambda i: (0,)),
                   pl.BlockSpec((t,), lambda i: (0,)),
                   pl.BlockSpec((nblk,), lambda i: (0,))],
        out_shape=[
            jax.ShapeDtypeStruct((t,), jnp.int32),
            jax.ShapeDtypeStruct((t,), jnp.int32),
            jax.ShapeDtypeStruct((nblk,), jnp.int32),
        ],
    )(oa, ob, ls2048)

    pos_all = jnp.concatenate([posa, posb])
    gate_all = jnp.concatenate([wa, wb])
    xg, rgate = _sc_dispatch(pos_all, gate_all, m, nrows=nrows, rpt=rpt, t=t,
                             dim=dim, nslots=nslots)

    y = pl.pallas_call(
        _gffn_body,
        grid_spec=pltpu.PrefetchScalarGridSpec(
            num_scalar_prefetch=1,
            grid=(nblk,),
            in_specs=[
                pl.BlockSpec((BLKF, dim), lambda bi, be_s: (bi, 0)),
                pl.BlockSpec((1, dim, hid),
                             lambda bi, be_s: (jnp.maximum(be_s[bi], 0), 0, 0)),
                pl.BlockSpec((1, dim, hid),
                             lambda bi, be_s: (jnp.maximum(be_s[bi], 0), 0, 0)),
                pl.BlockSpec((1, hid, dim),
                             lambda bi, be_s: (jnp.maximum(be_s[bi], 0), 0, 0)),
                pl.BlockSpec((BLKF,), lambda bi, be_s: (bi,)),
            ],
            out_specs=pl.BlockSpec((BLKF, dim), lambda bi, be_s: (bi, 0)),
        ),
        out_shape=jax.ShapeDtypeStruct((nrows, dim), bf),
        compiler_params=pltpu.CompilerParams(
            dimension_semantics=("arbitrary",)),
    )(be, xg, wg_b, wi_b, woe_b, rgate)

    yg = _sc_combine(pos_all, y, spt=spt, dim=dim, nslots=nslots)
    ya, yb = yg[:t], yg[t:]

    out = pl.pallas_call(
        _final_body,
        grid=(nt,),
        in_specs=[rowblk(dim), rowblk(dim), rowblk(dim)],
        out_specs=rowblk(dim),
        out_shape=jax.ShapeDtypeStruct((t, dim), f32),
    )(x2, ya, yb)

    return out.reshape(b, t, dim)
